# baseline (speedup 1.0000x reference)
import functools

import jax
import jax.numpy as jnp
from jax import lax
from jax.experimental import pallas as pl
from jax.experimental.pallas import tpu as pltpu
from jax.experimental.pallas import tpu_sc as plsc

N = 10000
D = 128
E = 320000
H = 64

NC = 2
NS = 16
NW = NC * NS
PER_W = E // NW
B = 128
NB = 80
NPASS = 1
NBP = NB // NPASS
PER_W_PAD = NB * B
PAD_PER_W = PER_W_PAD - PER_W
ZROWS = 632
ACC_ROWS = NS * ZROWS
OROWS = 624
TAIL = N - NS * OROWS


def _sc_aggregate(x, src3, dst3, zrows):
  mesh = plsc.VectorSubcoreMesh(core_axis_name="c", subcore_axis_name="s")

  @functools.partial(
      pl.kernel,
      out_type=jax.ShapeDtypeStruct((NC, N, D), jnp.float32),
      mesh=mesh,
      scratch_types=[
          pltpu.VMEM((NBP, B), jnp.int32),
          pltpu.VMEM((NBP, B), jnp.int32),
          pltpu.VMEM((B, D), jnp.float32),
          pltpu.VMEM_SHARED((ACC_ROWS, D), jnp.float32),
          pltpu.SemaphoreType.DMA,
      ],
  )
  def agg(x_hbm, src_hbm, dst_hbm, z_hbm, out_hbm, src_v, dst_v, rows_v,
          acc_sh, sem):
    c = lax.axis_index("c")
    s = lax.axis_index("s")
    w = s * NC + c
    pltpu.sync_copy(z_hbm, acc_sh.at[pl.ds(s * ZROWS, ZROWS)])
    plsc.subcore_barrier()

    for p in range(NPASS):
      pltpu.sync_copy(src_hbm.at[w, p], src_v)
      pltpu.sync_copy(dst_hbm.at[w, p], dst_v)

      def body(j, carry):
        pltpu.async_copy(x_hbm.at[src_v.at[j]], rows_v, sem).wait()
        pltpu.sync_copy(rows_v, acc_sh.at[dst_v.at[j]], add=True)
        return carry

      lax.fori_loop(0, NBP, body, 0)
    plsc.subcore_barrier()
    pltpu.sync_copy(acc_sh.at[pl.ds(s * OROWS, OROWS)],
                    out_hbm.at[c, pl.ds(s * OROWS, OROWS)])

    @pl.when(s == 0)
    def _():
      pltpu.sync_copy(acc_sh.at[pl.ds(NS * OROWS, TAIL)],
                      out_hbm.at[c, pl.ds(NS * OROWS, TAIL)])

  return agg(x, src3, dst3, zrows)


def _elu(v):
  return jnp.where(v > 0, v, jnp.exp(v) - 1.0)


def _tc_dense(partials, wg, bg, we1, be1, we2, be2, wd1, bd1, wd2, bd2):
  BLK = 1000

  def body(p_ref, wg_ref, bg_ref, we1_ref, be1_ref, we2_ref, be2_ref,
           wd1_ref, bd1_ref, wd2_ref, bd2_ref, o_ref):
    hb = p_ref[0] + p_ref[1]
    g3 = jnp.dot(hb, wg_ref[...], preferred_element_type=jnp.float32)
    g3 = g3 + bg_ref[...]
    ig = jax.nn.sigmoid(g3[:, :H])
    gg = jnp.tanh(g3[:, H:2 * H])
    og = jax.nn.sigmoid(g3[:, 2 * H:])
    h = jax.nn.relu(og * jnp.tanh(ig * gg))
    e = _elu(jnp.dot(h, we1_ref[...], preferred_element_type=jnp.float32)
             + be1_ref[...])
    e = _elu(jnp.dot(e, we2_ref[...], preferred_element_type=jnp.float32)
             + be2_ref[...])
    d = _elu(jnp.dot(e, wd1_ref[...], preferred_element_type=jnp.float32)
             + bd1_ref[...])
    o_ref[...] = _elu(
        jnp.dot(d, wd2_ref[...], preferred_element_type=jnp.float32)
        + bd2_ref[...])

  const = lambda i: (0, 0)
  return pl.pallas_call(
      body,
      grid=(N // BLK,),
      in_specs=[
          pl.BlockSpec((NC, BLK, D), lambda i: (0, i, 0)),
          pl.BlockSpec((D, 3 * H), const),
          pl.BlockSpec((1, 3 * H), const),
          pl.BlockSpec((H, 256), const),
          pl.BlockSpec((1, 256), const),
          pl.BlockSpec((256, 128), const),
          pl.BlockSpec((1, 128), const),
          pl.BlockSpec((128, 256), const),
          pl.BlockSpec((1, 256), const),
          pl.BlockSpec((256, H), const),
          pl.BlockSpec((1, H), const),
      ],
      out_specs=pl.BlockSpec((BLK, H), lambda i: (i, 0)),
      out_shape=jax.ShapeDtypeStruct((N, H), jnp.float32),
  )(partials, wg, bg, we1, be1, we2, be2, wd1, bd1, wd2, bd2)


def kernel(inputs, edge_index, W_ih, W_hh, b_ih, b_hh,
           W_e1, b_e1, W_e2, b_e2, W_d1, b_d1, W_d2, b_d2):
  src = edge_index[0].astype(jnp.int32)
  dst = edge_index[1].astype(jnp.int32)
  src3 = jnp.concatenate(
      [src.reshape(NW, PER_W),
       jnp.zeros((NW, PAD_PER_W), jnp.int32)], axis=1).reshape(NW, NPASS, NBP, B)
  pad_dst = N + (jnp.arange(PAD_PER_W, dtype=jnp.int32) % (ACC_ROWS - N))
  dst3 = jnp.concatenate(
      [dst.reshape(NW, PER_W),
       jnp.broadcast_to(pad_dst, (NW, PAD_PER_W))], axis=1).reshape(
           NW, NPASS, NBP, B)
  zrows = jnp.zeros((ZROWS, D), jnp.float32)

  partials = _sc_aggregate(inputs, src3, dst3, zrows)

  bsum = b_ih + b_hh
  wg = jnp.concatenate(
      [W_ih[0:H], W_ih[2 * H:3 * H], W_ih[3 * H:4 * H]], axis=0).T
  bg = jnp.concatenate(
      [bsum[0:H], bsum[2 * H:3 * H], bsum[3 * H:4 * H]]).reshape(1, 3 * H)

  return _tc_dense(
      partials,
      wg, bg,
      W_e1.T, b_e1.reshape(1, -1),
      W_e2.T, b_e2.reshape(1, -1),
      W_d1.T, b_d1.reshape(1, -1),
      W_d2.T, b_d2.reshape(1, -1))


# speedup vs baseline: 1.4348x; 1.4348x over previous
import functools

import jax
import jax.numpy as jnp
from jax import lax
from jax.experimental import pallas as pl
from jax.experimental.pallas import tpu as pltpu
from jax.experimental.pallas import tpu_sc as plsc

N = 10000
D = 128
E = 320000
H = 64

NC = 2
NS = 16
NW = NC * NS
PER_W = E // NW
B = 128
NB = 79
NPASS = 1
NBP = NB // NPASS
PER_W_PAD = NB * B
PAD_PER_W = PER_W_PAD - PER_W
ZROWS = 632
ACC_ROWS = NS * ZROWS
OROWS = 624
TAIL = N - NS * OROWS


def _sc_aggregate(x, src3, dst3, zrows):
  mesh = plsc.VectorSubcoreMesh(core_axis_name="c", subcore_axis_name="s")

  @functools.partial(
      pl.kernel,
      out_type=jax.ShapeDtypeStruct((NC, N, D), jnp.float32),
      mesh=mesh,
      scratch_types=[
          pltpu.VMEM((NBP, B), jnp.int32),
          pltpu.VMEM((NBP, B), jnp.int32),
          pltpu.VMEM((B, D), jnp.float32),
          pltpu.VMEM_SHARED((ACC_ROWS, D), jnp.float32),
          pltpu.SemaphoreType.DMA,
      ],
  )
  def agg(x_hbm, src_hbm, dst_hbm, z_hbm, out_hbm, src_v, dst_v, rows_v,
          acc_sh, sem):
    c = lax.axis_index("c")
    s = lax.axis_index("s")
    w = s * NC + c
    pltpu.sync_copy(z_hbm, acc_sh.at[pl.ds(s * ZROWS, ZROWS)])
    plsc.subcore_barrier()

    pltpu.sync_copy(src_hbm.at[w], src_v)
    pltpu.sync_copy(dst_hbm.at[w], dst_v)

    def body(j, carry):
      pltpu.async_copy(x_hbm.at[src_v.at[j]], rows_v, sem).wait()
      pltpu.sync_copy(rows_v, acc_sh.at[dst_v.at[j]], add=True)
      return carry

    lax.fori_loop(0, NB, body, 0)
    plsc.subcore_barrier()
    pltpu.sync_copy(acc_sh.at[pl.ds(s * OROWS, OROWS)],
                    out_hbm.at[c, pl.ds(s * OROWS, OROWS)])

    @pl.when(s == 0)
    def _():
      pltpu.sync_copy(acc_sh.at[pl.ds(NS * OROWS, TAIL)],
                      out_hbm.at[c, pl.ds(NS * OROWS, TAIL)])

  return agg(x, src3, dst3, zrows)


def _elu(v):
  return jnp.where(v > 0, v, jnp.exp(v) - 1.0)


def _tc_dense(partials, wg, bg, we1, be1, we2, be2, wd1, bd1, wd2, bd2):
  BLK = 1000

  def body(p_ref, wg_ref, bg_ref, we1_ref, be1_ref, we2_ref, be2_ref,
           wd1_ref, bd1_ref, wd2_ref, bd2_ref, o_ref):
    hb = p_ref[0] + p_ref[1]
    g3 = jnp.dot(hb, wg_ref[...], preferred_element_type=jnp.float32)
    g3 = g3 + bg_ref[...]
    ig = jax.nn.sigmoid(g3[:, :H])
    gg = jnp.tanh(g3[:, H:2 * H])
    og = jax.nn.sigmoid(g3[:, 2 * H:])
    h = jax.nn.relu(og * jnp.tanh(ig * gg))
    e = _elu(jnp.dot(h, we1_ref[...], preferred_element_type=jnp.float32)
             + be1_ref[...])
    e = _elu(jnp.dot(e, we2_ref[...], preferred_element_type=jnp.float32)
             + be2_ref[...])
    d = _elu(jnp.dot(e, wd1_ref[...], preferred_element_type=jnp.float32)
             + bd1_ref[...])
    o_ref[...] = _elu(
        jnp.dot(d, wd2_ref[...], preferred_element_type=jnp.float32)
        + bd2_ref[...])

  const = lambda i: (0, 0)
  return pl.pallas_call(
      body,
      grid=(N // BLK,),
      in_specs=[
          pl.BlockSpec((NC, BLK, D), lambda i: (0, i, 0)),
          pl.BlockSpec((D, 3 * H), const),
          pl.BlockSpec((1, 3 * H), const),
          pl.BlockSpec((H, 256), const),
          pl.BlockSpec((1, 256), const),
          pl.BlockSpec((256, 128), const),
          pl.BlockSpec((1, 128), const),
          pl.BlockSpec((128, 256), const),
          pl.BlockSpec((1, 256), const),
          pl.BlockSpec((256, H), const),
          pl.BlockSpec((1, H), const),
      ],
      out_specs=pl.BlockSpec((BLK, H), lambda i: (i, 0)),
      out_shape=jax.ShapeDtypeStruct((N, H), jnp.float32),
  )(partials, wg, bg, we1, be1, we2, be2, wd1, bd1, wd2, bd2)


def kernel(inputs, edge_index, W_ih, W_hh, b_ih, b_hh,
           W_e1, b_e1, W_e2, b_e2, W_d1, b_d1, W_d2, b_d2):
  src = edge_index[0].astype(jnp.int32)
  dst = edge_index[1].astype(jnp.int32)
  src3 = jnp.concatenate(
      [src.reshape(NW, PER_W),
       jnp.zeros((NW, PAD_PER_W), jnp.int32)], axis=1).reshape(NW, NB, B)
  pad_dst = N + (jnp.arange(PAD_PER_W, dtype=jnp.int32) % (ACC_ROWS - N))
  dst3 = jnp.concatenate(
      [dst.reshape(NW, PER_W),
       jnp.broadcast_to(pad_dst, (NW, PAD_PER_W))], axis=1).reshape(NW, NB, B)
  zrows = jnp.zeros((ZROWS, D), jnp.float32)

  partials = _sc_aggregate(inputs, src3, dst3, zrows)

  bsum = b_ih + b_hh
  wg = jnp.concatenate(
      [W_ih[0:H], W_ih[2 * H:3 * H], W_ih[3 * H:4 * H]], axis=0).T
  bg = jnp.concatenate(
      [bsum[0:H], bsum[2 * H:3 * H], bsum[3 * H:4 * H]]).reshape(1, 3 * H)

  return _tc_dense(
      partials,
      wg, bg,
      W_e1.T, b_e1.reshape(1, -1),
      W_e2.T, b_e2.reshape(1, -1),
      W_d1.T, b_d1.reshape(1, -1),
      W_d2.T, b_d2.reshape(1, -1))


# speedup vs baseline: 2.2207x; 1.5477x over previous
import functools

import jax
import jax.numpy as jnp
from jax import lax
from jax.experimental import pallas as pl
from jax.experimental.pallas import tpu as pltpu
from jax.experimental.pallas import tpu_sc as plsc

N = 10000
D = 128
E = 320000
H = 64

NC = 2
NS = 16
NW = NC * NS
PER_W = E // NW
B = 128
NB = 79
NPASS = 1
NBP = NB // NPASS
PER_W_PAD = NB * B
PAD_PER_W = PER_W_PAD - PER_W
ZROWS = 632
ACC_ROWS = NS * ZROWS
OROWS = 624
TAIL = N - NS * OROWS


def _sc_aggregate(x, src3, dst3, zrows):
  mesh = plsc.VectorSubcoreMesh(core_axis_name="c", subcore_axis_name="s")

  @functools.partial(
      pl.kernel,
      out_type=jax.ShapeDtypeStruct((NC, N, D), jnp.float32),
      mesh=mesh,
      scratch_types=[
          pltpu.VMEM((NBP, B), jnp.int32),
          pltpu.VMEM((NBP, B), jnp.int32),
          pltpu.VMEM((B, D), jnp.float32),
          pltpu.VMEM_SHARED((ACC_ROWS, D), jnp.float32),
          pltpu.SemaphoreType.DMA,
      ],
  )
  def agg(x_hbm, src_hbm, dst_hbm, z_hbm, out_hbm, src_v, dst_v, rows_v,
          acc_sh, sem):
    c = lax.axis_index("c")
    s = lax.axis_index("s")
    w = s * NC + c
    pltpu.sync_copy(z_hbm, acc_sh.at[pl.ds(s * ZROWS, ZROWS)])
    plsc.subcore_barrier()

    pltpu.sync_copy(src_hbm.at[w], src_v)
    pltpu.sync_copy(dst_hbm.at[w], dst_v)

    def body(j, carry):
      pltpu.async_copy(x_hbm.at[src_v.at[j]], rows_v, sem).wait()
      pltpu.sync_copy(rows_v, acc_sh.at[dst_v.at[j]], add=True)
      return carry

    lax.fori_loop(0, NB, body, 0)
    plsc.subcore_barrier()
    pltpu.sync_copy(acc_sh.at[pl.ds(s * OROWS, OROWS)],
                    out_hbm.at[c, pl.ds(s * OROWS, OROWS)])

    @pl.when(s == 0)
    def _():
      pltpu.sync_copy(acc_sh.at[pl.ds(NS * OROWS, TAIL)],
                      out_hbm.at[c, pl.ds(NS * OROWS, TAIL)])

  return agg(x, src3, dst3, zrows)


def _elu(v):
  return jnp.where(v > 0, v, jnp.exp(v) - 1.0)


def _tc_dense(partials, wg, bg, we1, be1, we2, be2, wd1, bd1, wd2, bd2):
  BLK = 1000

  def body(p_ref, wg_ref, bg_ref, we1_ref, be1_ref, we2_ref, be2_ref,
           wd1_ref, bd1_ref, wd2_ref, bd2_ref, o_ref):
    hb = p_ref[0] + p_ref[1]
    g3 = jnp.dot(hb, wg_ref[...], preferred_element_type=jnp.float32)
    g3 = g3 + bg_ref[...]
    ig = jax.nn.sigmoid(g3[:, :H])
    gg = jnp.tanh(g3[:, H:2 * H])
    og = jax.nn.sigmoid(g3[:, 2 * H:])
    h = jax.nn.relu(og * jnp.tanh(ig * gg))
    e = _elu(jnp.dot(h, we1_ref[...], preferred_element_type=jnp.float32)
             + be1_ref[...])
    e = _elu(jnp.dot(e, we2_ref[...], preferred_element_type=jnp.float32)
             + be2_ref[...])
    d = _elu(jnp.dot(e, wd1_ref[...], preferred_element_type=jnp.float32)
             + bd1_ref[...])
    o_ref[...] = _elu(
        jnp.dot(d, wd2_ref[...], preferred_element_type=jnp.float32)
        + bd2_ref[...])

  const = lambda i: (0, 0)
  return pl.pallas_call(
      body,
      grid=(N // BLK,),
      in_specs=[
          pl.BlockSpec((NC, BLK, D), lambda i: (0, i, 0)),
          pl.BlockSpec((D, 3 * H), const),
          pl.BlockSpec((1, 3 * H), const),
          pl.BlockSpec((H, 256), const),
          pl.BlockSpec((1, 256), const),
          pl.BlockSpec((256, 128), const),
          pl.BlockSpec((1, 128), const),
          pl.BlockSpec((128, 256), const),
          pl.BlockSpec((1, 256), const),
          pl.BlockSpec((256, H), const),
          pl.BlockSpec((1, H), const),
      ],
      out_specs=pl.BlockSpec((BLK, H), lambda i: (i, 0)),
      out_shape=jax.ShapeDtypeStruct((N, H), jnp.float32),
  )(partials, wg, bg, we1, be1, we2, be2, wd1, bd1, wd2, bd2)


def kernel(inputs, edge_index, W_ih, W_hh, b_ih, b_hh,
           W_e1, b_e1, W_e2, b_e2, W_d1, b_d1, W_d2, b_d2):
  src = edge_index[0].astype(jnp.int32)
  dst = edge_index[1].astype(jnp.int32)
  pad_src = jnp.arange(PAD_PER_W, dtype=jnp.int32) % N
  src3 = jnp.concatenate(
      [src.reshape(NW, PER_W),
       jnp.broadcast_to(pad_src, (NW, PAD_PER_W))], axis=1).reshape(NW, NB, B)
  pad_dst = N + (jnp.arange(PAD_PER_W, dtype=jnp.int32) % (ACC_ROWS - N))
  dst3 = jnp.concatenate(
      [dst.reshape(NW, PER_W),
       jnp.broadcast_to(pad_dst, (NW, PAD_PER_W))], axis=1).reshape(NW, NB, B)
  zrows = jnp.zeros((ZROWS, D), jnp.float32)

  partials = _sc_aggregate(inputs, src3, dst3, zrows)

  bsum = b_ih + b_hh
  wg = jnp.concatenate(
      [W_ih[0:H], W_ih[2 * H:3 * H], W_ih[3 * H:4 * H]], axis=0).T
  bg = jnp.concatenate(
      [bsum[0:H], bsum[2 * H:3 * H], bsum[3 * H:4 * H]]).reshape(1, 3 * H)

  return _tc_dense(
      partials,
      wg, bg,
      W_e1.T, b_e1.reshape(1, -1),
      W_e2.T, b_e2.reshape(1, -1),
      W_d1.T, b_d1.reshape(1, -1),
      W_d2.T, b_d2.reshape(1, -1))


# speedup vs baseline: 3.0521x; 1.3743x over previous
import functools

import jax
import jax.numpy as jnp
from jax import lax
from jax.experimental import pallas as pl
from jax.experimental.pallas import tpu as pltpu
from jax.experimental.pallas import tpu_sc as plsc

N = 10000
D = 128
E = 320000
H = 64

NC = 2
NS = 16
NW = NC * NS
PER_W = E // NW
B = 128
NB = 80
NPASS = 2
NBP = NB // NPASS
PER_W_PAD = NB * B
PAD_PER_W = PER_W_PAD - PER_W
ZROWS = 632
ACC_ROWS = NS * ZROWS
OROWS = 624
TAIL = N - NS * OROWS


def _sc_aggregate(x, src3, dst3, zrows):
  mesh = plsc.VectorSubcoreMesh(core_axis_name="c", subcore_axis_name="s")

  @functools.partial(
      pl.kernel,
      out_type=jax.ShapeDtypeStruct((NC, N, D), jnp.float32),
      mesh=mesh,
      scratch_types=[
          pltpu.VMEM((NBP, B), jnp.int32),
          pltpu.VMEM((NBP, B), jnp.int32),
          pltpu.VMEM((B, D), jnp.float32),
          pltpu.VMEM((B, D), jnp.float32),
          pltpu.VMEM_SHARED((ACC_ROWS, D), jnp.float32),
          pltpu.SemaphoreType.DMA,
          pltpu.SemaphoreType.DMA,
      ],
  )
  def agg(x_hbm, src_hbm, dst_hbm, z_hbm, out_hbm, src_v, dst_v, rows0_v,
          rows1_v, acc_sh, sem0, sem1):
    c = lax.axis_index("c")
    s = lax.axis_index("s")
    w = s * NC + c
    pltpu.sync_copy(z_hbm, acc_sh.at[pl.ds(s * ZROWS, ZROWS)])
    plsc.subcore_barrier()

    for p in range(NPASS):
      pltpu.sync_copy(src_hbm.at[w, p], src_v)
      pltpu.sync_copy(dst_hbm.at[w, p], dst_v)

      pltpu.async_copy(x_hbm.at[src_v.at[0]], rows0_v, sem0)
      pltpu.async_copy(x_hbm.at[src_v.at[1]], rows1_v, sem1)

      def body(k, carry):
        j0 = 2 * k
        j1 = 2 * k + 1
        pltpu.make_async_copy(x_hbm.at[src_v.at[j0]], rows0_v, sem0).wait()
        pltpu.sync_copy(rows0_v, acc_sh.at[dst_v.at[j0]], add=True)
        pltpu.async_copy(x_hbm.at[src_v.at[j0 + 2]], rows0_v, sem0)
        pltpu.make_async_copy(x_hbm.at[src_v.at[j1]], rows1_v, sem1).wait()
        pltpu.sync_copy(rows1_v, acc_sh.at[dst_v.at[j1]], add=True)
        pltpu.async_copy(x_hbm.at[src_v.at[j1 + 2]], rows1_v, sem1)
        return carry

      lax.fori_loop(0, NBP // 2 - 1, body, 0)
      pltpu.make_async_copy(x_hbm.at[src_v.at[NBP - 2]], rows0_v, sem0).wait()
      pltpu.sync_copy(rows0_v, acc_sh.at[dst_v.at[NBP - 2]], add=True)
      pltpu.make_async_copy(x_hbm.at[src_v.at[NBP - 1]], rows1_v, sem1).wait()
      pltpu.sync_copy(rows1_v, acc_sh.at[dst_v.at[NBP - 1]], add=True)
    plsc.subcore_barrier()
    pltpu.sync_copy(acc_sh.at[pl.ds(s * OROWS, OROWS)],
                    out_hbm.at[c, pl.ds(s * OROWS, OROWS)])

    @pl.when(s == 0)
    def _():
      pltpu.sync_copy(acc_sh.at[pl.ds(NS * OROWS, TAIL)],
                      out_hbm.at[c, pl.ds(NS * OROWS, TAIL)])

  return agg(x, src3, dst3, zrows)


def _elu(v):
  return jnp.where(v > 0, v, jnp.exp(v) - 1.0)


def _tc_dense(partials, wg, bg, we1, be1, we2, be2, wd1, bd1, wd2, bd2):
  BLK = 1000

  def body(p_ref, wg_ref, bg_ref, we1_ref, be1_ref, we2_ref, be2_ref,
           wd1_ref, bd1_ref, wd2_ref, bd2_ref, o_ref):
    hb = p_ref[0] + p_ref[1]
    g3 = jnp.dot(hb, wg_ref[...], preferred_element_type=jnp.float32)
    g3 = g3 + bg_ref[...]
    ig = jax.nn.sigmoid(g3[:, :H])
    gg = jnp.tanh(g3[:, H:2 * H])
    og = jax.nn.sigmoid(g3[:, 2 * H:])
    h = jax.nn.relu(og * jnp.tanh(ig * gg))
    e = _elu(jnp.dot(h, we1_ref[...], preferred_element_type=jnp.float32)
             + be1_ref[...])
    e = _elu(jnp.dot(e, we2_ref[...], preferred_element_type=jnp.float32)
             + be2_ref[...])
    d = _elu(jnp.dot(e, wd1_ref[...], preferred_element_type=jnp.float32)
             + bd1_ref[...])
    o_ref[...] = _elu(
        jnp.dot(d, wd2_ref[...], preferred_element_type=jnp.float32)
        + bd2_ref[...])

  const = lambda i: (0, 0)
  return pl.pallas_call(
      body,
      grid=(N // BLK,),
      in_specs=[
          pl.BlockSpec((NC, BLK, D), lambda i: (0, i, 0)),
          pl.BlockSpec((D, 3 * H), const),
          pl.BlockSpec((1, 3 * H), const),
          pl.BlockSpec((H, 256), const),
          pl.BlockSpec((1, 256), const),
          pl.BlockSpec((256, 128), const),
          pl.BlockSpec((1, 128), const),
          pl.BlockSpec((128, 256), const),
          pl.BlockSpec((1, 256), const),
          pl.BlockSpec((256, H), const),
          pl.BlockSpec((1, H), const),
      ],
      out_specs=pl.BlockSpec((BLK, H), lambda i: (i, 0)),
      out_shape=jax.ShapeDtypeStruct((N, H), jnp.float32),
  )(partials, wg, bg, we1, be1, we2, be2, wd1, bd1, wd2, bd2)


def kernel(inputs, edge_index, W_ih, W_hh, b_ih, b_hh,
           W_e1, b_e1, W_e2, b_e2, W_d1, b_d1, W_d2, b_d2):
  src = edge_index[0].astype(jnp.int32)
  dst = edge_index[1].astype(jnp.int32)
  pad_src = jnp.arange(PAD_PER_W, dtype=jnp.int32) % N
  src3 = jnp.concatenate(
      [src.reshape(NW, PER_W),
       jnp.broadcast_to(pad_src, (NW, PAD_PER_W))], axis=1).reshape(
           NW, NPASS, NBP, B)
  pad_dst = N + (jnp.arange(PAD_PER_W, dtype=jnp.int32) % (ACC_ROWS - N))
  dst3 = jnp.concatenate(
      [dst.reshape(NW, PER_W),
       jnp.broadcast_to(pad_dst, (NW, PAD_PER_W))], axis=1).reshape(
           NW, NPASS, NBP, B)
  zrows = jnp.zeros((ZROWS, D), jnp.float32)

  partials = _sc_aggregate(inputs, src3, dst3, zrows)

  bsum = b_ih + b_hh
  wg = jnp.concatenate(
      [W_ih[0:H], W_ih[2 * H:3 * H], W_ih[3 * H:4 * H]], axis=0).T
  bg = jnp.concatenate(
      [bsum[0:H], bsum[2 * H:3 * H], bsum[3 * H:4 * H]]).reshape(1, 3 * H)

  return _tc_dense(
      partials,
      wg, bg,
      W_e1.T, b_e1.reshape(1, -1),
      W_e2.T, b_e2.reshape(1, -1),
      W_d1.T, b_d1.reshape(1, -1),
      W_d2.T, b_d2.reshape(1, -1))
